# baseline (device time: 56470 ns/iter reference)
import jax
import jax.numpy as jnp
from jax import lax
from jax.experimental import pallas as pl
from jax.experimental.pallas import tpu as pltpu

N_DEV = 16


def kernel(x, dest):
    t, n = x.shape

    def body(
        dest_ref,
        x_ref,
        out_ref,
        my_counts,
        lrank,
        counts_all,
        base,
        cnt_send_sems,
        cnt_recv_sems,
        data_send_sem,
        data_recv_sem,
        local_sem,
    ):
        my_pos = lax.axis_index("i")

        barrier_sem = pltpu.get_barrier_semaphore()
        for p in range(N_DEV):

            @pl.when(p != my_pos)
            def _():
                pl.semaphore_signal(
                    barrier_sem,
                    inc=1,
                    device_id=(p,),
                    device_id_type=pl.DeviceIdType.MESH,
                )

        pl.semaphore_wait(barrier_sem, N_DEV - 1)

        for d in range(N_DEV):
            my_counts[d] = 0

        def count_body(j, c):
            d = dest_ref[j]
            lrank[j] = my_counts[d]
            my_counts[d] = my_counts[d] + 1
            return c

        lax.fori_loop(0, t, count_body, 0)

        for d in range(N_DEV):
            counts_all[my_pos, d] = my_counts[d]

        for p in range(N_DEV):

            @pl.when(p != my_pos)
            def _():
                pltpu.make_async_remote_copy(
                    src_ref=my_counts,
                    dst_ref=counts_all.at[my_pos],
                    send_sem=cnt_send_sems.at[p],
                    recv_sem=cnt_recv_sems.at[my_pos],
                    device_id=(p,),
                    device_id_type=pl.DeviceIdType.MESH,
                ).start()

        for p in range(N_DEV):

            @pl.when(p != my_pos)
            def _():
                rdma = pltpu.make_async_remote_copy(
                    src_ref=my_counts,
                    dst_ref=counts_all.at[p],
                    send_sem=cnt_send_sems.at[p],
                    recv_sem=cnt_recv_sems.at[p],
                    device_id=(p,),
                    device_id_type=pl.DeviceIdType.MESH,
                )
                rdma.wait_send()
                rdma.wait_recv()

        for d in range(N_DEV):
            base[d] = 0

        def base_body(s, c):
            for d in range(N_DEV):
                base[d] = base[d] + counts_all[s, d]
            return c

        lax.fori_loop(0, my_pos, base_body, 0)

        def send_body(j, c):
            d = dest_ref[j]
            row = base[d] + lrank[j]

            @pl.when(d == my_pos)
            def _():
                pltpu.make_async_copy(
                    x_ref.at[pl.ds(j, 1)],
                    out_ref.at[pl.ds(row, 1)],
                    local_sem,
                ).start()

            @pl.when(d != my_pos)
            def _():
                pltpu.make_async_remote_copy(
                    src_ref=x_ref.at[pl.ds(j, 1)],
                    dst_ref=out_ref.at[pl.ds(row, 1)],
                    send_sem=data_send_sem,
                    recv_sem=data_recv_sem,
                    device_id=(d,),
                    device_id_type=pl.DeviceIdType.MESH,
                ).start()

            return c

        lax.fori_loop(0, t, send_body, 0)

        n_local = my_counts[my_pos]
        nrecv = jnp.int32(0)
        for p in range(N_DEV):
            nrecv = nrecv + jnp.where(
                p != my_pos, counts_all[p, my_pos], 0
            )

        def wait_local_body(j, c):
            pltpu.make_async_copy(
                x_ref.at[pl.ds(0, 1)],
                out_ref.at[pl.ds(0, 1)],
                local_sem,
            ).wait()
            return c

        lax.fori_loop(0, n_local, wait_local_body, 0)

        def wait_send_body(j, c):
            pltpu.make_async_remote_copy(
                src_ref=x_ref.at[pl.ds(0, 1)],
                dst_ref=out_ref.at[pl.ds(0, 1)],
                send_sem=data_send_sem,
                recv_sem=data_recv_sem,
                device_id=(my_pos,),
                device_id_type=pl.DeviceIdType.MESH,
            ).wait_send()
            return c

        lax.fori_loop(0, t - n_local, wait_send_body, 0)

        def wait_recv_body(j, c):
            pltpu.make_async_remote_copy(
                src_ref=x_ref.at[pl.ds(0, 1)],
                dst_ref=out_ref.at[pl.ds(0, 1)],
                send_sem=data_send_sem,
                recv_sem=data_recv_sem,
                device_id=(my_pos,),
                device_id_type=pl.DeviceIdType.MESH,
            ).wait_recv()
            return c

        lax.fori_loop(0, nrecv, wait_recv_body, 0)

        def exit_barrier(sem):
            for p in range(N_DEV):

                @pl.when(p != my_pos)
                def _():
                    pl.semaphore_signal(
                        sem,
                        inc=1,
                        device_id=(p,),
                        device_id_type=pl.DeviceIdType.MESH,
                    )

            pl.semaphore_wait(sem, N_DEV - 1)

        pl.run_scoped(exit_barrier, sem=pltpu.SemaphoreType.REGULAR)

    return pl.pallas_call(
        body,
        out_shape=jax.ShapeDtypeStruct((t, n), x.dtype),
        in_specs=[
            pl.BlockSpec(memory_space=pltpu.SMEM),
            pl.BlockSpec(memory_space=pltpu.VMEM),
        ],
        out_specs=pl.BlockSpec(memory_space=pltpu.VMEM),
        scratch_shapes=[
            pltpu.SMEM((N_DEV,), jnp.int32),
            pltpu.SMEM((t,), jnp.int32),
            pltpu.SMEM((N_DEV, N_DEV), jnp.int32),
            pltpu.SMEM((N_DEV,), jnp.int32),
            pltpu.SemaphoreType.DMA((N_DEV,)),
            pltpu.SemaphoreType.DMA((N_DEV,)),
            pltpu.SemaphoreType.DMA,
            pltpu.SemaphoreType.DMA,
            pltpu.SemaphoreType.DMA,
        ],
        compiler_params=pltpu.CompilerParams(collective_id=0),
    )(dest, x)
